# baseline (speedup 1.0000x reference)
import functools

import jax
import jax.numpy as jnp
from jax import lax
from jax.experimental import pallas as pl
from jax.experimental.pallas import tpu as pltpu
from jax.experimental.pallas import tpu_sc as plsc

NUM_GRIDS = 8
GS = 64
FEAT = 16
NPTS = 524288
NC = 2
NSUB = 16
L = 16
NW = NC * NSUB
PW = NPTS // NW
C = 128
G = C // L
NCHUNK = PW // C
TABLE_ROWS = NUM_GRIDS * GS * GS * GS


def _worker_id():
    return lax.axis_index("s") * NC + lax.axis_index("c")


def _compute_chunk(c, pbuf, idxbuf, wbuf):
    for k in range(G):
        sl = pl.ds(c * C + k * L, L)
        lo = []
        step = []
        w0 = []
        w1 = []
        inside = None
        for d in range(3):
            v = pbuf[d, sl]
            p = (v + 1.0) * 0.5
            s = p * float(GS - 1)
            b = s.astype(jnp.int32)
            f = s - b.astype(jnp.float32)
            b0 = jnp.clip(b, 0, GS - 1)
            b1 = jnp.clip(b + 1, 0, GS - 1)
            lo.append(b0)
            step.append(b1 - b0)
            w0.append(1.0 - f)
            w1.append(f)
            ok = (p >= 0.0) & (p <= 1.0)
            inside = ok if inside is None else (inside & ok)
        m = jnp.where(inside, 1.0, 0.0).astype(jnp.float32)
        w0[2] = w0[2] * m
        w1[2] = w1[2] * m

        gi = pbuf[3, sl].astype(jnp.int32)
        base_row = (gi * (GS * GS * GS)
                    + lo[0] * (GS * GS) + lo[1] * GS + lo[2])
        dx = step[0] * (GS * GS)
        dy = step[1] * GS
        dz = step[2]

        wxy = [w0[0] * w0[1], w1[0] * w0[1], w0[0] * w1[1], w1[0] * w1[1]]
        for j in range(8):
            off = base_row
            if j & 1:
                off = off + dx
            if j & 2:
                off = off + dy
            if j & 4:
                off = off + dz
            idxbuf[k, pl.ds(j * L, L)] = off
            wj = wxy[j & 3] * (w1[2] if j & 4 else w0[2])
            wbuf[k, pl.ds(j * L, L)] = wj


def _fire_gather(table, idxbuf, rows, gsem):
    for k in range(G):
        pltpu.async_copy(
            table.at[idxbuf.at[k]], rows.at[pl.ds(k * 8 * L, 8 * L)], gsem
        )


def _wait_gather(table, rows, gsem):
    pltpu.make_async_copy(table.at[pl.ds(0, C * 8)], rows, gsem).wait()


def _accumulate_chunk(wbuf, rows, outbuf):
    def acc_group(k, carry):
        wvs = [wbuf[k, pl.ds(j * L, L)] for j in range(8)]
        for p in range(L):
            acc = None
            for j in range(8):
                w = wvs[j][p]
                row = rows[k * 8 * L + j * L + p, :]
                acc = w * row if acc is None else acc + w * row
            outbuf[k * L + p, :] = acc
        return carry

    lax.fori_loop(0, G, acc_group, 0, unroll=False)


def _stage_body(ve, out, buf0, buf1, isem0, isem1, osem0, osem1):
    wid = _worker_id()
    g = wid // 4
    xbase = (wid % 4) * 16
    bufs = (buf0, buf1)
    isems = (isem0, isem1)
    osems = (osem0, osem1)
    nch = 32
    hy = GS // 2

    def fill(i, b):
        x = xbase + i // 2
        y0 = (i % 2) * hy
        for y in range(hy):
            pltpu.async_copy(
                ve.at[g, x, y0 + y], bufs[b].at[pl.ds(y * GS, GS)], isems[b]
            )

    def dst_rows(i):
        x = xbase + i // 2
        y0 = (i % 2) * hy
        r = ((g * GS + x) * GS + y0) * GS
        return out.at[pl.ds(r, hy * GS)]

    def drain_fill(b):
        pltpu.make_async_copy(out.at[pl.ds(0, hy * GS)], bufs[b], isems[b]).wait()

    fill(0, 0)
    for i in range(nch):
        b = i % 2
        drain_fill(b)
        pltpu.async_copy(bufs[b], dst_rows(i), osems[b])
        if i + 1 < nch:
            if i >= 1:
                pltpu.make_async_copy(bufs[1 - b], dst_rows(i - 1), osems[1 - b]).wait()
            fill(i + 1, 1 - b)
    pltpu.make_async_copy(bufs[0], dst_rows(nch - 2), osems[0]).wait()
    pltpu.make_async_copy(bufs[1], dst_rows(nch - 1), osems[1]).wait()


def _sc_body(table, packed, out, pbuf, idxbufs, wbufs, rowss,
             outbufs, gsems, osems):
    wid = _worker_id()
    base = wid * PW

    pltpu.sync_copy(packed.at[:, pl.ds(base, PW)], pbuf)

    _compute_chunk(0, pbuf, idxbufs[0], wbufs[0])
    _fire_gather(table, idxbufs[0], rowss[0], gsems[0])

    def pair(cc, carry):
        for b in range(2):
            c = cc * 2 + b
            nb = 1 - b

            @pl.when(c + 1 < NCHUNK)
            def _prefetch():
                _compute_chunk(c + 1, pbuf, idxbufs[nb], wbufs[nb])
                _fire_gather(table, idxbufs[nb], rowss[nb], gsems[nb])

            _wait_gather(table, rowss[b], gsems[b])

            @pl.when(c >= 2)
            def _drain_out():
                pltpu.make_async_copy(
                    outbufs[b], out.at[pl.ds(base + (c - 2) * C, C)], osems[b]
                ).wait()

            _accumulate_chunk(wbufs[b], rowss[b], outbufs[b])
            pltpu.async_copy(
                outbufs[b], out.at[pl.ds(base + c * C, C)], osems[b]
            )
        return carry

    lax.fori_loop(0, NCHUNK // 2, pair, 0, unroll=False)

    for b in range(2):
        c = NCHUNK - 2 + b
        pltpu.make_async_copy(
            outbufs[b], out.at[pl.ds(base + c * C, C)], osems[b]
        ).wait()


def _body(table, packed, out,
          pbuf, idxbuf0, idxbuf1, wbuf0, wbuf1, rows0, rows1,
          outbuf0, outbuf1, gsem0, gsem1, osem0, osem1):
    _sc_body(table, packed, out, pbuf,
             (idxbuf0, idxbuf1), (wbuf0, wbuf1), (rows0, rows1),
             (outbuf0, outbuf1), (gsem0, gsem1), (osem0, osem1))


_scratch = [
    pltpu.VMEM((4, PW), jnp.float32),
    pltpu.VMEM((G, 8 * L), jnp.int32),
    pltpu.VMEM((G, 8 * L), jnp.int32),
    pltpu.VMEM((G, 8 * L), jnp.float32),
    pltpu.VMEM((G, 8 * L), jnp.float32),
    pltpu.VMEM((C * 8, FEAT), jnp.float32),
    pltpu.VMEM((C * 8, FEAT), jnp.float32),
    pltpu.VMEM((C, FEAT), jnp.float32),
    pltpu.VMEM((C, FEAT), jnp.float32),
    pltpu.SemaphoreType.DMA,
    pltpu.SemaphoreType.DMA,
    pltpu.SemaphoreType.DMA,
    pltpu.SemaphoreType.DMA,
]

_mesh = plsc.VectorSubcoreMesh(
    core_axis_name="c", subcore_axis_name="s", num_cores=NC, num_subcores=NSUB
)

_sc_stage = pl.kernel(
    _stage_body,
    out_type=jax.ShapeDtypeStruct((TABLE_ROWS, FEAT), jnp.float32),
    mesh=_mesh,
    scratch_types=[
        pltpu.VMEM((GS // 2 * GS, FEAT), jnp.float32),
        pltpu.VMEM((GS // 2 * GS, FEAT), jnp.float32),
        pltpu.SemaphoreType.DMA,
        pltpu.SemaphoreType.DMA,
        pltpu.SemaphoreType.DMA,
        pltpu.SemaphoreType.DMA,
    ],
    compiler_params=pltpu.CompilerParams(
        use_tc_tiling_on_sc=False, needs_layout_passes=False
    ),
)

_sc_interp = pl.kernel(
    _body,
    out_type=jax.ShapeDtypeStruct((NPTS, FEAT), jnp.float32),
    mesh=_mesh,
    scratch_types=_scratch,
    compiler_params=pltpu.CompilerParams(use_tc_tiling_on_sc=False),
)


@jax.jit
def kernel(voxel_embeddings, grid_indexes, points):
    table = _sc_stage(voxel_embeddings)
    packed = jnp.concatenate(
        [points.T, grid_indexes.reshape(1, NPTS).astype(jnp.float32)], axis=0
    )
    return _sc_interp(table, packed)


# speedup vs baseline: 1.0439x; 1.0439x over previous
import functools

import jax
import jax.numpy as jnp
from jax import lax
from jax.experimental import pallas as pl
from jax.experimental.pallas import tpu as pltpu
from jax.experimental.pallas import tpu_sc as plsc

NUM_GRIDS = 8
GS = 64
FEAT = 16
NPTS = 524288
NC = 2
NSUB = 16
L = 16
NW = NC * NSUB
PW = NPTS // NW
C = 128
G = C // L
NCHUNK = PW // C
TABLE_ROWS = NUM_GRIDS * GS * GS * GS


def _worker_id():
    return lax.axis_index("s") * NC + lax.axis_index("c")


def _compute_chunk(c, pbuf, idxbuf, wbuf):
    for k in range(G):
        sl = pl.ds(c * C + k * L, L)
        lo = []
        step = []
        w0 = []
        w1 = []
        inside = None
        for d in range(3):
            v = pbuf[d, sl]
            p = (v + 1.0) * 0.5
            s = p * float(GS - 1)
            b = s.astype(jnp.int32)
            f = s - b.astype(jnp.float32)
            b0 = jnp.clip(b, 0, GS - 1)
            b1 = jnp.clip(b + 1, 0, GS - 1)
            lo.append(b0)
            step.append(b1 - b0)
            w0.append(1.0 - f)
            w1.append(f)
            ok = (p >= 0.0) & (p <= 1.0)
            inside = ok if inside is None else (inside & ok)
        m = jnp.where(inside, 1.0, 0.0).astype(jnp.float32)
        w0[2] = w0[2] * m
        w1[2] = w1[2] * m

        gi = pbuf[3, sl].astype(jnp.int32)
        base_row = (gi * (GS * GS * GS)
                    + lo[0] * (GS * GS) + lo[1] * GS + lo[2])
        dx = step[0] * (GS * GS)
        dy = step[1] * GS
        dz = step[2]

        wxy = [w0[0] * w0[1], w1[0] * w0[1], w0[0] * w1[1], w1[0] * w1[1]]
        for j in range(8):
            off = base_row
            if j & 1:
                off = off + dx
            if j & 2:
                off = off + dy
            if j & 4:
                off = off + dz
            idxbuf[k, pl.ds(j * L, L)] = off
            wj = wxy[j & 3] * (w1[2] if j & 4 else w0[2])
            wbuf[k, pl.ds(j * L, L)] = wj


def _fire_gather(table, idxbuf, rows, gsem):
    for k in range(G):
        pltpu.async_copy(
            table.at[idxbuf.at[k]], rows.at[pl.ds(k * 8 * L, 8 * L)], gsem
        )


def _wait_gather(table, rows, gsem):
    pltpu.make_async_copy(table.at[pl.ds(0, C * 8)], rows, gsem).wait()


def _accumulate_chunk(wbuf, rows, outbuf):
    def acc_group(k, carry):
        wvs = [wbuf[k, pl.ds(j * L, L)] for j in range(8)]
        for p in range(L):
            acc = None
            for j in range(8):
                w = wvs[j][p]
                row = rows[k * 8 * L + j * L + p, :]
                acc = w * row if acc is None else acc + w * row
            outbuf[k * L + p, :] = acc
        return carry

    lax.fori_loop(0, G, acc_group, 0, unroll=False)


def _stage_body(ve, out, buf0, buf1, isem0, isem1, osem0, osem1):
    wid = _worker_id()
    g = wid // 4
    xbase = (wid % 4) * 16
    bufs = (buf0, buf1)
    isems = (isem0, isem1)
    osems = (osem0, osem1)
    nch = 32
    hy = GS // 2

    def fill(i, b):
        x = xbase + i // 2
        y0 = (i % 2) * hy
        for y in range(hy):
            pltpu.async_copy(
                ve.at[g, x, y0 + y], bufs[b].at[pl.ds(y * GS, GS)], isems[b]
            )

    def dst_rows(i):
        x = xbase + i // 2
        y0 = (i % 2) * hy
        r = ((g * GS + x) * GS + y0) * GS
        return out.at[pl.ds(r, hy * GS)]

    def drain_fill(b):
        pltpu.make_async_copy(out.at[pl.ds(0, hy * GS)], bufs[b], isems[b]).wait()

    fill(0, 0)
    for i in range(nch):
        b = i % 2
        drain_fill(b)
        pltpu.async_copy(bufs[b], dst_rows(i), osems[b])
        if i + 1 < nch:
            if i >= 1:
                pltpu.make_async_copy(bufs[1 - b], dst_rows(i - 1), osems[1 - b]).wait()
            fill(i + 1, 1 - b)
    pltpu.make_async_copy(bufs[0], dst_rows(nch - 2), osems[0]).wait()
    pltpu.make_async_copy(bufs[1], dst_rows(nch - 1), osems[1]).wait()


def _sc_body(table, packed, out, pbuf, idxbufs, wbufs, rowss,
             outbufs, gsems, osems):
    wid = _worker_id()
    base = wid * PW

    pltpu.sync_copy(packed.at[:, pl.ds(base, PW)], pbuf)

    for c0 in range(2):
        _compute_chunk(c0, pbuf, idxbufs[c0], wbufs[c0])
        _fire_gather(table, idxbufs[c0], rowss[c0], gsems[c0])

    ntrip = (NCHUNK + 2) // 3

    def trip(cc, carry):
        for b in range(3):
            c = cc * 3 + b

            @pl.when(c < NCHUNK)
            def _chunk():
                b2 = (b + 2) % 3

                @pl.when(c + 2 < NCHUNK)
                def _prefetch():
                    _compute_chunk(c + 2, pbuf, idxbufs[b2], wbufs[b2])
                    _fire_gather(table, idxbufs[b2], rowss[b2], gsems[b2])

                _wait_gather(table, rowss[b], gsems[b])

                @pl.when(c >= 3)
                def _drain_out():
                    pltpu.make_async_copy(
                        outbufs[b], out.at[pl.ds(base + (c - 3) * C, C)],
                        osems[b],
                    ).wait()

                _accumulate_chunk(wbufs[b], rowss[b], outbufs[b])
                pltpu.async_copy(
                    outbufs[b], out.at[pl.ds(base + c * C, C)], osems[b]
                )
        return carry

    lax.fori_loop(0, ntrip, trip, 0, unroll=False)

    for c in range(NCHUNK - 3, NCHUNK):
        b = c % 3
        pltpu.make_async_copy(
            outbufs[b], out.at[pl.ds(base + c * C, C)], osems[b]
        ).wait()


def _body(table, packed, out,
          pbuf, idxbuf0, idxbuf1, idxbuf2, wbuf0, wbuf1, wbuf2,
          rows0, rows1, rows2, outbuf0, outbuf1, outbuf2,
          gsem0, gsem1, gsem2, osem0, osem1, osem2):
    _sc_body(table, packed, out, pbuf,
             (idxbuf0, idxbuf1, idxbuf2), (wbuf0, wbuf1, wbuf2),
             (rows0, rows1, rows2), (outbuf0, outbuf1, outbuf2),
             (gsem0, gsem1, gsem2), (osem0, osem1, osem2))


_scratch = (
    [pltpu.VMEM((4, PW), jnp.float32)]
    + [pltpu.VMEM((G, 8 * L), jnp.int32) for _ in range(3)]
    + [pltpu.VMEM((G, 8 * L), jnp.float32) for _ in range(3)]
    + [pltpu.VMEM((C * 8, FEAT), jnp.float32) for _ in range(3)]
    + [pltpu.VMEM((C, FEAT), jnp.float32) for _ in range(3)]
    + [pltpu.SemaphoreType.DMA for _ in range(6)]
)

_mesh = plsc.VectorSubcoreMesh(
    core_axis_name="c", subcore_axis_name="s", num_cores=NC, num_subcores=NSUB
)

_sc_stage = pl.kernel(
    _stage_body,
    out_type=jax.ShapeDtypeStruct((TABLE_ROWS, FEAT), jnp.float32),
    mesh=_mesh,
    scratch_types=[
        pltpu.VMEM((GS // 2 * GS, FEAT), jnp.float32),
        pltpu.VMEM((GS // 2 * GS, FEAT), jnp.float32),
        pltpu.SemaphoreType.DMA,
        pltpu.SemaphoreType.DMA,
        pltpu.SemaphoreType.DMA,
        pltpu.SemaphoreType.DMA,
    ],
    compiler_params=pltpu.CompilerParams(
        use_tc_tiling_on_sc=False, needs_layout_passes=False
    ),
)

_sc_interp = pl.kernel(
    _body,
    out_type=jax.ShapeDtypeStruct((NPTS, FEAT), jnp.float32),
    mesh=_mesh,
    scratch_types=_scratch,
    compiler_params=pltpu.CompilerParams(use_tc_tiling_on_sc=False),
)


@jax.jit
def kernel(voxel_embeddings, grid_indexes, points):
    table = voxel_embeddings.reshape(TABLE_ROWS, FEAT)
    packed = jnp.concatenate(
        [points.T, grid_indexes.reshape(1, NPTS).astype(jnp.float32)], axis=0
    )
    return _sc_interp(table, packed)


# speedup vs baseline: 1.0846x; 1.0389x over previous
import functools

import jax
import jax.numpy as jnp
from jax import lax
from jax.experimental import pallas as pl
from jax.experimental.pallas import tpu as pltpu
from jax.experimental.pallas import tpu_sc as plsc

NUM_GRIDS = 8
GS = 64
FEAT = 16
NPTS = 524288
NC = 2
NSUB = 16
L = 16
NW = NC * NSUB
PW = NPTS // NW
C = 128
G = C // L
NCHUNK = PW // C
TABLE_ROWS = NUM_GRIDS * GS * GS * GS


def _worker_id():
    return lax.axis_index("s") * NC + lax.axis_index("c")


def _compute_chunk(c, pbuf, idxbuf, wbuf):
    for k in range(G):
        sl = pl.ds(c * C + k * L, L)
        lo = []
        step = []
        w0 = []
        w1 = []
        inside = None
        for d in range(3):
            v = pbuf[d, sl]
            p = (v + 1.0) * 0.5
            s = p * float(GS - 1)
            b = s.astype(jnp.int32)
            f = s - b.astype(jnp.float32)
            b0 = jnp.clip(b, 0, GS - 1)
            b1 = jnp.clip(b + 1, 0, GS - 1)
            lo.append(b0)
            step.append(b1 - b0)
            w0.append(1.0 - f)
            w1.append(f)
            ok = (p >= 0.0) & (p <= 1.0)
            inside = ok if inside is None else (inside & ok)
        m = jnp.where(inside, 1.0, 0.0).astype(jnp.float32)
        w0[2] = w0[2] * m
        w1[2] = w1[2] * m

        gi = pbuf[3, sl].astype(jnp.int32)
        base_row = (gi * (GS * GS * GS)
                    + lo[0] * (GS * GS) + lo[1] * GS + lo[2])
        dx = step[0] * (GS * GS)
        dy = step[1] * GS
        dz = step[2]

        wxy = [w0[0] * w0[1], w1[0] * w0[1], w0[0] * w1[1], w1[0] * w1[1]]
        for j in range(8):
            off = base_row
            if j & 1:
                off = off + dx
            if j & 2:
                off = off + dy
            if j & 4:
                off = off + dz
            idxbuf[pl.ds((k * 8 + j) * L, L)] = off
            wj = wxy[j & 3] * (w1[2] if j & 4 else w0[2])
            wbuf[k, pl.ds(j * L, L)] = wj


def _fire_gather(table, idxbuf, rows, gsem):
    pltpu.async_copy(table.at[idxbuf], rows, gsem)


def _wait_gather(table, rows, gsem):
    pltpu.make_async_copy(table.at[pl.ds(0, C * 8)], rows, gsem).wait()


def _accumulate_chunk(wbuf, rows, outbuf):
    def acc_group(k, carry):
        wvs = [wbuf[k, pl.ds(j * L, L)] for j in range(8)]
        for p in range(L):
            acc = None
            for j in range(8):
                w = wvs[j][p]
                row = rows[k * 8 * L + j * L + p, :]
                acc = w * row if acc is None else acc + w * row
            outbuf[k * L + p, :] = acc
        return carry

    lax.fori_loop(0, G, acc_group, 0, unroll=False)


def _stage_body(ve, out, buf0, buf1, isem0, isem1, osem0, osem1):
    wid = _worker_id()
    g = wid // 4
    xbase = (wid % 4) * 16
    bufs = (buf0, buf1)
    isems = (isem0, isem1)
    osems = (osem0, osem1)
    nch = 32
    hy = GS // 2

    def fill(i, b):
        x = xbase + i // 2
        y0 = (i % 2) * hy
        for y in range(hy):
            pltpu.async_copy(
                ve.at[g, x, y0 + y], bufs[b].at[pl.ds(y * GS, GS)], isems[b]
            )

    def dst_rows(i):
        x = xbase + i // 2
        y0 = (i % 2) * hy
        r = ((g * GS + x) * GS + y0) * GS
        return out.at[pl.ds(r, hy * GS)]

    def drain_fill(b):
        pltpu.make_async_copy(out.at[pl.ds(0, hy * GS)], bufs[b], isems[b]).wait()

    fill(0, 0)
    for i in range(nch):
        b = i % 2
        drain_fill(b)
        pltpu.async_copy(bufs[b], dst_rows(i), osems[b])
        if i + 1 < nch:
            if i >= 1:
                pltpu.make_async_copy(bufs[1 - b], dst_rows(i - 1), osems[1 - b]).wait()
            fill(i + 1, 1 - b)
    pltpu.make_async_copy(bufs[0], dst_rows(nch - 2), osems[0]).wait()
    pltpu.make_async_copy(bufs[1], dst_rows(nch - 1), osems[1]).wait()


def _sc_body(table, packed, out, pbuf, idxbufs, wbufs, rowss,
             outbufs, gsems, osems):
    wid = _worker_id()
    base = wid * PW

    pltpu.sync_copy(packed.at[:, pl.ds(base, PW)], pbuf)

    _compute_chunk(0, pbuf, idxbufs[0], wbufs[0])
    _fire_gather(table, idxbufs[0], rowss[0], gsems[0])

    def pair(cc, carry):
        for b in range(2):
            c = cc * 2 + b
            nb = 1 - b

            @pl.when(c + 1 < NCHUNK)
            def _prefetch():
                _compute_chunk(c + 1, pbuf, idxbufs[nb], wbufs[nb])
                _fire_gather(table, idxbufs[nb], rowss[nb], gsems[nb])

            _wait_gather(table, rowss[b], gsems[b])

            @pl.when(c >= 2)
            def _drain_out():
                pltpu.make_async_copy(
                    outbufs[b], out.at[pl.ds(base + (c - 2) * C, C)], osems[b]
                ).wait()

            _accumulate_chunk(wbufs[b], rowss[b], outbufs[b])
            pltpu.async_copy(
                outbufs[b], out.at[pl.ds(base + c * C, C)], osems[b]
            )
        return carry

    lax.fori_loop(0, NCHUNK // 2, pair, 0, unroll=False)

    for b in range(2):
        c = NCHUNK - 2 + b
        pltpu.make_async_copy(
            outbufs[b], out.at[pl.ds(base + c * C, C)], osems[b]
        ).wait()


def _body(table, packed, out,
          pbuf, idxbuf0, idxbuf1, wbuf0, wbuf1, rows0, rows1,
          outbuf0, outbuf1, gsem0, gsem1, osem0, osem1):
    _sc_body(table, packed, out, pbuf,
             (idxbuf0, idxbuf1), (wbuf0, wbuf1), (rows0, rows1),
             (outbuf0, outbuf1), (gsem0, gsem1), (osem0, osem1))


_scratch = (
    [pltpu.VMEM((4, PW), jnp.float32)]
    + [pltpu.VMEM((G * 8 * L,), jnp.int32) for _ in range(2)]
    + [pltpu.VMEM((G, 8 * L), jnp.float32) for _ in range(2)]
    + [pltpu.VMEM((C * 8, FEAT), jnp.float32) for _ in range(2)]
    + [pltpu.VMEM((C, FEAT), jnp.float32) for _ in range(2)]
    + [pltpu.SemaphoreType.DMA for _ in range(4)]
)

_mesh = plsc.VectorSubcoreMesh(
    core_axis_name="c", subcore_axis_name="s", num_cores=NC, num_subcores=NSUB
)

_sc_stage = pl.kernel(
    _stage_body,
    out_type=jax.ShapeDtypeStruct((TABLE_ROWS, FEAT), jnp.float32),
    mesh=_mesh,
    scratch_types=[
        pltpu.VMEM((GS // 2 * GS, FEAT), jnp.float32),
        pltpu.VMEM((GS // 2 * GS, FEAT), jnp.float32),
        pltpu.SemaphoreType.DMA,
        pltpu.SemaphoreType.DMA,
        pltpu.SemaphoreType.DMA,
        pltpu.SemaphoreType.DMA,
    ],
    compiler_params=pltpu.CompilerParams(
        use_tc_tiling_on_sc=False, needs_layout_passes=False
    ),
)

_sc_interp = pl.kernel(
    _body,
    out_type=jax.ShapeDtypeStruct((NPTS, FEAT), jnp.float32),
    mesh=_mesh,
    scratch_types=_scratch,
    compiler_params=pltpu.CompilerParams(use_tc_tiling_on_sc=False),
)


@jax.jit
def kernel(voxel_embeddings, grid_indexes, points):
    table = voxel_embeddings.reshape(TABLE_ROWS, FEAT)
    packed = jnp.concatenate(
        [points.T, grid_indexes.reshape(1, NPTS).astype(jnp.float32)], axis=0
    )
    return _sc_interp(table, packed)
